# Initial kernel scaffold; baseline (speedup 1.0000x reference)
import functools

import jax
import jax.numpy as jnp
from jax import lax
from jax.experimental import pallas as pl
from jax.experimental.pallas import tpu as pltpu
from jax.experimental.pallas import tpu_sc as plsc

N = 10000
E = 320000
D = 128

NC = 2
NS = 16
NW = NC * NS

CH = 128
EPW = -(-E // (NW * CH)) * CH
E_PAD = EPW * NW
RPT = -(-N // (NS * CH)) * CH
N_PAD = RPT * NS

BR = 512


@functools.partial(
    pl.kernel,
    out_type=jax.ShapeDtypeStruct((NC, N_PAD, D), jnp.float32),
    mesh=plsc.VectorSubcoreMesh(core_axis_name="c", subcore_axis_name="s"),
    scratch_types=[
        pltpu.VMEM((CH,), jnp.int32),
        pltpu.VMEM((CH,), jnp.int32),
        pltpu.VMEM((CH, D), jnp.float32),
        pltpu.VMEM_SHARED((N_PAD, D), jnp.float32),
        pltpu.SemaphoreType.DMA,
    ],
)
def _sc_aggregate(feat_hbm, src_hbm, dst_hbm, out_hbm, sidx, didx, rows, agg,
                  sem):
    c = lax.axis_index("c")
    s = lax.axis_index("s")
    r0 = s * RPT

    def init_i(i, carry):
        off = r0 + i * CH
        pltpu.sync_copy(feat_hbm.at[pl.ds(off, CH)], rows)
        pltpu.sync_copy(rows, agg.at[pl.ds(off, CH)])
        return carry

    lax.fori_loop(0, RPT // CH, init_i, 0)
    plsc.subcore_barrier()

    base = (s * NC + c) * EPW

    def edge_i(i, carry):
        off = base + i * CH
        pltpu.sync_copy(src_hbm.at[pl.ds(off, CH)], sidx)
        pltpu.sync_copy(dst_hbm.at[pl.ds(off, CH)], didx)
        pltpu.async_copy(feat_hbm.at[sidx], rows, sem).wait()
        pltpu.sync_copy(rows, agg.at[didx], add=True)
        return carry

    lax.fori_loop(0, EPW // CH, edge_i, 0)
    plsc.subcore_barrier()

    def dump_i(i, carry):
        off = r0 + i * CH
        pltpu.sync_copy(agg.at[pl.ds(off, CH)], rows)
        pltpu.sync_copy(rows, out_hbm.at[c, pl.ds(off, CH)])
        return carry

    lax.fori_loop(0, RPT // CH, dump_i, 0)


def _tc_body(f_ref, a0_ref, a1_ref, wt_ref, b_ref, o_ref):
    x = a0_ref[0] + a1_ref[0] - f_ref[...]
    y = jnp.dot(x, wt_ref[...], preferred_element_type=jnp.float32)
    o_ref[...] = jnp.maximum(y + b_ref[...], 0.0)


_tc_linear = pl.pallas_call(
    _tc_body,
    grid=(pl.cdiv(N, BR),),
    in_specs=[
        pl.BlockSpec((BR, D), lambda i: (i, 0)),
        pl.BlockSpec((1, BR, D), lambda i: (0, i, 0)),
        pl.BlockSpec((1, BR, D), lambda i: (1, i, 0)),
        pl.BlockSpec((D, D), lambda i: (0, 0)),
        pl.BlockSpec((1, D), lambda i: (0, 0)),
    ],
    out_specs=pl.BlockSpec((BR, D), lambda i: (i, 0)),
    out_shape=jax.ShapeDtypeStruct((N, D), jnp.float32),
)


def kernel(feature, edge_index, W, b):
    pad_e = E_PAD - E
    src = jnp.concatenate([edge_index[0], jnp.zeros((pad_e,), jnp.int32)])
    dst = jnp.concatenate(
        [edge_index[1], jnp.full((pad_e,), N, jnp.int32)])
    feat_p = jnp.concatenate(
        [feature, jnp.zeros((N_PAD - N, D), jnp.float32)])
    agg = _sc_aggregate(feat_p, src, dst)
    return _tc_linear(feature, agg, agg, W.T, b.reshape(1, D))


# speedup vs baseline: 3.8911x; 3.8911x over previous
import functools

import jax
import jax.numpy as jnp
from jax import lax
from jax.experimental import pallas as pl
from jax.experimental.pallas import tpu as pltpu
from jax.experimental.pallas import tpu_sc as plsc

N = 10000
E = 320000
D = 128

NC = 2
NS = 16
NW = NC * NS

CH = 128
EPW = -(-E // (NW * CH)) * CH
E_PAD = EPW * NW
RPT = -(-N // (NS * CH)) * CH
N_PAD = RPT * NS

BR = 512


def _sc_aggregate_body(feat_hbm, src_hbm, dst_hbm, out_hbm, sidx, didx, rows,
                       agg, sem):
    c = lax.axis_index("c")
    s = lax.axis_index("s")
    r0 = s * RPT

    def init_i(i, carry):
        off = r0 + i * CH
        pltpu.sync_copy(feat_hbm.at[pl.ds(off, CH)], rows)
        pltpu.sync_copy(rows, agg.at[pl.ds(off, CH)])
        return carry

    lax.fori_loop(0, RPT // CH, init_i, 0)
    plsc.subcore_barrier()

    base = (s * NC + c) * EPW

    def edge_i(i, carry):
        off = base + i * CH
        pltpu.sync_copy(src_hbm.at[pl.ds(off, CH)], sidx)
        pltpu.sync_copy(dst_hbm.at[pl.ds(off, CH)], didx)
        pltpu.async_copy(feat_hbm.at[sidx], rows, sem).wait()
        pltpu.sync_copy(rows, agg.at[didx], add=True)
        return carry

    lax.fori_loop(0, EPW // CH, edge_i, 0)
    plsc.subcore_barrier()

    def dump_i(i, carry):
        off = r0 + i * CH
        pltpu.sync_copy(agg.at[pl.ds(off, CH)], rows)
        pltpu.sync_copy(rows, out_hbm.at[c, pl.ds(off, CH)])
        return carry

    lax.fori_loop(0, RPT // CH, dump_i, 0)


@functools.cache
def _sc_aggregate():
    return pl.kernel(
        _sc_aggregate_body,
        out_type=jax.ShapeDtypeStruct((NC, N_PAD, D), jnp.float32),
        mesh=plsc.VectorSubcoreMesh(core_axis_name="c", subcore_axis_name="s"),
        scratch_types=[
            pltpu.VMEM((CH,), jnp.int32),
            pltpu.VMEM((CH,), jnp.int32),
            pltpu.VMEM((CH, D), jnp.float32),
            pltpu.VMEM_SHARED((N_PAD, D), jnp.float32),
            pltpu.SemaphoreType.DMA,
        ],
    )


def _tc_body(f_ref, a0_ref, a1_ref, wt_ref, b_ref, o_ref):
    x = a0_ref[0] + a1_ref[0] - f_ref[...]
    y = jnp.dot(x, wt_ref[...], preferred_element_type=jnp.float32)
    o_ref[...] = jnp.maximum(y + b_ref[...], 0.0)


_tc_linear = pl.pallas_call(
    _tc_body,
    grid=(pl.cdiv(N, BR),),
    in_specs=[
        pl.BlockSpec((BR, D), lambda i: (i, 0)),
        pl.BlockSpec((1, BR, D), lambda i: (0, i, 0)),
        pl.BlockSpec((1, BR, D), lambda i: (1, i, 0)),
        pl.BlockSpec((D, D), lambda i: (0, 0)),
        pl.BlockSpec((1, D), lambda i: (0, 0)),
    ],
    out_specs=pl.BlockSpec((BR, D), lambda i: (i, 0)),
    out_shape=jax.ShapeDtypeStruct((N, D), jnp.float32),
)


def kernel(feature, edge_index, W, b):
    pad_e = E_PAD - E
    src = jnp.concatenate([edge_index[0], jnp.zeros((pad_e,), jnp.int32)])
    dst = jnp.concatenate(
        [edge_index[1], jnp.full((pad_e,), N, jnp.int32)])
    feat_p = jnp.concatenate(
        [feature, jnp.zeros((N_PAD - N, D), jnp.float32)])
    agg = _sc_aggregate()(feat_p, src, dst)
    return _tc_linear(feature, agg, agg, W.T, b.reshape(1, D))
